# baseline (speedup 1.0000x reference)
import functools
import math

import jax
import jax.numpy as jnp
import numpy as np
from jax import lax
from jax.experimental import pallas as pl
from jax.experimental.pallas import tpu as pltpu

_FCT = 5

_U32 = np.uint32


def _rotl32(x, r):
    return (x << _U32(r)) | (x >> _U32(32 - r))


def _threefry2x32(ks0, ks1, x0, x1):
    x0 = np.asarray(x0, _U32).copy()
    x1 = np.asarray(x1, _U32).copy()
    ks2 = _U32(ks0 ^ ks1 ^ _U32(0x1BD11BDA))
    ks = [_U32(ks0), _U32(ks1), ks2]
    rots = ((13, 15, 26, 6), (17, 29, 16, 24))
    x0 = x0 + ks[0]
    x1 = x1 + ks[1]
    for i in range(5):
        for r in rots[i % 2]:
            x0 = x0 + x1
            x1 = _rotl32(x1, r)
            x1 = x1 ^ x0
        x0 = x0 + ks[(i + 1) % 3]
        x1 = x1 + ks[(i + 2) % 3] + _U32(i + 1)
    return x0, x1


def _np_randint(seed, shape, minval, maxval):
    k0 = _U32(np.uint64(seed) >> np.uint64(32))
    k1 = _U32(np.uint64(seed) & np.uint64(0xFFFFFFFF))
    o1, o2 = _threefry2x32(k0, k1, np.zeros(2, _U32), np.arange(2, dtype=_U32))
    n = int(np.prod(shape))

    def bits(ka, kb):
        b1, b2 = _threefry2x32(ka, kb, np.zeros(n, _U32),
                               np.arange(n, dtype=_U32))
        return b1 ^ b2

    hi_b = bits(o1[0], o2[0])
    lo_b = bits(o1[1], o2[1])
    span = int(maxval - minval)
    mult = _U32((2 ** 16 % span) ** 2 % span)
    off = ((hi_b % _U32(span)) * mult + lo_b % _U32(span)) % _U32(span)
    return (minval + off.astype(np.int64)).reshape(shape)


def _pair_kernel(q_ref, k_ref, v_ref, cnt_hbm, cntT_hbm, o_ref,
                 cnt_ref, cntT_ref, sem1, sem2, *,
                 L, D, U, U_pad, CH):
    f32 = jnp.float32
    hi = lax.Precision.HIGHEST
    df = lax.Precision.DEFAULT

    step = pl.program_id(0) * pl.num_programs(1) + pl.program_id(1)

    @pl.when(step == 0)
    def _stage():
        c1 = pltpu.make_async_copy(cnt_hbm, cnt_ref, sem1)
        c2 = pltpu.make_async_copy(cntT_hbm, cntT_ref, sem2)
        c1.start()
        c2.start()
        c1.wait()
        c2.wait()

    q2 = q_ref[...].reshape(L, 2 * D)
    k2 = k_ref[...].reshape(L, 2 * D)
    v2 = v_ref[...].reshape(L, 2 * D)

    kc2 = lax.dot_general(cnt_ref[...], k2, (((1,), (0,)), ((), ())),
                          preferred_element_type=f32, precision=df)
    qb2 = q2.astype(jnp.bfloat16).astype(f32)
    hsel = (lax.broadcasted_iota(jnp.int32, (2, 2 * D), 1) // D ==
            lax.broadcasted_iota(jnp.int32, (2, 2 * D), 0)).astype(f32)
    ssum2 = lax.dot_general(hsel, qb2 * kc2, (((1,), (1,)), ((), ())),
                            preferred_element_type=f32, precision=hi)

    uf = jnp.float32(U)
    ri = lax.broadcasted_iota(jnp.int32, (U_pad, 1), 0).astype(f32)
    fr = (lax.broadcasted_iota(jnp.int32, (4, 1), 0).astype(f32) + 1.0) * 0.2

    ctxs = []
    for hh in range(2):
        sl = slice(hh * D, (hh + 1) * D)
        q = q2[:, sl]
        k = k2[:, sl]
        v = v2[:, sl]
        kb = k.astype(jnp.bfloat16)

        smax_parts = []
        for c in range(L // CH):
            qcb = q[c * CH:(c + 1) * CH, :].astype(jnp.bfloat16)
            s_t = lax.dot_general(kb, qcb, (((1,), (1,)), ((), ())),
                                  preferred_element_type=f32, precision=df)
            cT = cntT_ref[:, c * CH:(c + 1) * CH]
            smax_parts.append(jnp.max(
                jnp.where(cT > 0, s_t, -1e30), axis=0, keepdims=True))
        m = (jnp.concatenate(smax_parts, axis=1)
             - ssum2[hh:hh + 1, :] * (1.0 / L))

        m4 = jnp.concatenate([m, m, m, m], axis=0)
        lo = jnp.min(m, axis=1, keepdims=True) - 1.0
        bhi = jnp.max(m, axis=1, keepdims=True) + 1.0
        for _ in range(17):
            mids = lo + (bhi - lo) * fr
            cgt = jnp.sum((m4 >= mids).astype(f32), axis=1, keepdims=True)
            take = cgt >= uf
            lo = jnp.maximum(lo, jnp.max(
                jnp.where(take, mids, -3e38), axis=0, keepdims=True))
            bhi = jnp.minimum(bhi, jnp.min(
                jnp.where(take, 3e38, mids), axis=0, keepdims=True))

        mk = (m >= lo).astype(f32)
        incl = mk
        sh = 1
        while sh < L:
            incl = incl + jnp.concatenate(
                [jnp.zeros((1, sh), f32), incl[:, :L - sh]], axis=1)
            sh *= 2
        rk = incl - mk
        p = ((ri == rk) & (mk > 0) & (rk < uf)).astype(f32)

        qr = lax.dot_general(p, q, (((1,), (0,)), ((), ())),
                             preferred_element_type=f32, precision=df)
        st = lax.dot_general(qr, k, (((1,), (1,)), ((), ())),
                             preferred_element_type=f32, precision=df)
        st = st * (1.0 / math.sqrt(D))
        mt = jnp.max(st, axis=1, keepdims=True)
        e = jnp.exp(st - mt)
        a = e / jnp.sum(e, axis=1, keepdims=True)
        upd = lax.dot_general(a, v, (((1,), (0,)), ((), ())),
                              preferred_element_type=f32, precision=df)

        vmean = jnp.sum(v, axis=0, keepdims=True) * (1.0 / L)
        ctxs.append(vmean + lax.dot_general(
            p, upd - vmean, (((0,), (0,)), ((), ())),
            preferred_element_type=f32, precision=df))

    o_ref[...] = jnp.concatenate(ctxs, axis=1).reshape(o_ref.shape)


def kernel(queries, keys, values):
    B, L, H, D = queries.shape
    L_K = keys.shape[1]
    u_samp = min(_FCT * int(np.ceil(np.log(L_K))), L_K)
    U = min(_FCT * int(np.ceil(np.log(L))), L)
    U_pad = max(8, ((U + 63) // 64) * 64)
    CH = 512

    idx = _np_randint(42, (L, u_samp), 0, L_K)
    cnt_np = np.zeros((L, L_K), np.float32)
    np.add.at(cnt_np, (np.arange(L)[:, None], idx), 1.0)
    cnt = jnp.asarray(cnt_np, dtype=jnp.bfloat16)
    cntT = jnp.asarray(cnt_np.T, dtype=jnp.bfloat16)

    q3 = queries.reshape(B, L, H * D)
    k3 = keys.reshape(B, L, H * D)
    v3 = values.reshape(B, L, H * D)

    body = functools.partial(_pair_kernel, L=L, D=D, U=U, U_pad=U_pad, CH=CH)
    out = pl.pallas_call(
        body,
        grid=(B, H // 2),
        in_specs=[
            pl.BlockSpec((1, L, 2 * D), lambda b, j: (b, 0, j)),
            pl.BlockSpec((1, L, 2 * D), lambda b, j: (b, 0, j)),
            pl.BlockSpec((1, L, 2 * D), lambda b, j: (b, 0, j)),
            pl.BlockSpec(memory_space=pl.ANY),
            pl.BlockSpec(memory_space=pl.ANY),
        ],
        out_specs=pl.BlockSpec((1, L, 2 * D), lambda b, j: (b, 0, j)),
        out_shape=jax.ShapeDtypeStruct((B, L, H * D), jnp.float32),
        scratch_shapes=[
            pltpu.VMEM((L, L_K), jnp.bfloat16),
            pltpu.VMEM((L_K, L), jnp.bfloat16),
            pltpu.SemaphoreType.DMA,
            pltpu.SemaphoreType.DMA,
        ],
        compiler_params=pltpu.CompilerParams(
            dimension_semantics=("parallel", "parallel"),
        ),
    )(q3, k3, v3, cnt, cntT)
    return out.reshape(B, L, H, D)


# speedup vs baseline: 1.0739x; 1.0739x over previous
import functools
import math

import jax
import jax.numpy as jnp
import numpy as np
from jax import lax
from jax.experimental import pallas as pl
from jax.experimental.pallas import tpu as pltpu

_FCT = 5

_U32 = np.uint32


def _rotl32(x, r):
    return (x << _U32(r)) | (x >> _U32(32 - r))


def _threefry2x32(ks0, ks1, x0, x1):
    x0 = np.asarray(x0, _U32).copy()
    x1 = np.asarray(x1, _U32).copy()
    ks2 = _U32(ks0 ^ ks1 ^ _U32(0x1BD11BDA))
    ks = [_U32(ks0), _U32(ks1), ks2]
    rots = ((13, 15, 26, 6), (17, 29, 16, 24))
    x0 = x0 + ks[0]
    x1 = x1 + ks[1]
    for i in range(5):
        for r in rots[i % 2]:
            x0 = x0 + x1
            x1 = _rotl32(x1, r)
            x1 = x1 ^ x0
        x0 = x0 + ks[(i + 1) % 3]
        x1 = x1 + ks[(i + 2) % 3] + _U32(i + 1)
    return x0, x1


def _np_randint(seed, shape, minval, maxval):
    k0 = _U32(np.uint64(seed) >> np.uint64(32))
    k1 = _U32(np.uint64(seed) & np.uint64(0xFFFFFFFF))
    o1, o2 = _threefry2x32(k0, k1, np.zeros(2, _U32), np.arange(2, dtype=_U32))
    n = int(np.prod(shape))

    def bits(ka, kb):
        b1, b2 = _threefry2x32(ka, kb, np.zeros(n, _U32),
                               np.arange(n, dtype=_U32))
        return b1 ^ b2

    hi_b = bits(o1[0], o2[0])
    lo_b = bits(o1[1], o2[1])
    span = int(maxval - minval)
    mult = _U32((2 ** 16 % span) ** 2 % span)
    off = ((hi_b % _U32(span)) * mult + lo_b % _U32(span)) % _U32(span)
    return (minval + off.astype(np.int64)).reshape(shape)


def _pair_kernel(q_ref, k_ref, v_ref, cnt_hbm, cntT_hbm, o_ref,
                 cnt_ref, cntT_ref, sem1, sem2, *,
                 L, D, U, U_pad, CH):
    f32 = jnp.float32
    hi = lax.Precision.HIGHEST
    df = lax.Precision.DEFAULT

    step = pl.program_id(0) * pl.num_programs(1) + pl.program_id(1)

    @pl.when(step == 0)
    def _stage():
        c1 = pltpu.make_async_copy(cnt_hbm, cnt_ref, sem1)
        c2 = pltpu.make_async_copy(cntT_hbm, cntT_ref, sem2)
        c1.start()
        c2.start()
        c1.wait()
        c2.wait()

    q2 = q_ref[...].reshape(L, 2 * D)
    k2 = k_ref[...].reshape(L, 2 * D)
    v2 = v_ref[...].reshape(L, 2 * D)

    kc2 = lax.dot_general(cnt_ref[...], k2, (((1,), (0,)), ((), ())),
                          preferred_element_type=f32, precision=df)
    qb2 = q2.astype(jnp.bfloat16).astype(f32)
    hsel = (lax.broadcasted_iota(jnp.int32, (2, 2 * D), 1) // D ==
            lax.broadcasted_iota(jnp.int32, (2, 2 * D), 0)).astype(f32)
    ssum2 = lax.dot_general(hsel, qb2 * kc2, (((1,), (1,)), ((), ())),
                            preferred_element_type=f32, precision=hi)

    uf = jnp.float32(U)
    ri = lax.broadcasted_iota(jnp.int32, (U_pad, 1), 0).astype(f32)
    NT = 16

    m_rows = []
    for hh in range(2):
        sl = slice(hh * D, (hh + 1) * D)
        kb = k2[:, sl].astype(jnp.bfloat16)
        smax_parts = []
        for c in range(L // CH):
            qcb = q2[c * CH:(c + 1) * CH, sl].astype(jnp.bfloat16)
            s_t = lax.dot_general(kb, qcb, (((1,), (1,)), ((), ())),
                                  preferred_element_type=f32, precision=df)
            cT = cntT_ref[:, c * CH:(c + 1) * CH]
            smax_parts.append(jnp.max(
                jnp.where(cT > 0, s_t, -1e30), axis=0, keepdims=True))
        m_rows.append(jnp.concatenate(smax_parts, axis=1))
    m2 = jnp.concatenate(m_rows, axis=0) - ssum2 * (1.0 / L)

    mrep = jnp.concatenate([m2] * NT, axis=0)
    frp = ((lax.broadcasted_iota(jnp.int32, (2 * NT, 1), 0) // 2).astype(f32)
           + 1.0) * (1.0 / (NT + 1.0))
    lo = jnp.min(m2, axis=1, keepdims=True) - 1.0
    bhi = jnp.max(m2, axis=1, keepdims=True) + 1.0
    for _ in range(9):
        lop = jnp.concatenate([lo] * NT, axis=0)
        hip = jnp.concatenate([bhi] * NT, axis=0)
        mids = lop + (hip - lop) * frp
        cgt = jnp.sum((mrep >= mids).astype(f32), axis=1, keepdims=True)
        take = cgt >= uf
        cl = jnp.max(jnp.where(take, mids, -3e38).reshape(NT, 2),
                     axis=0, keepdims=True).reshape(2, 1)
        ch_ = jnp.min(jnp.where(take, 3e38, mids).reshape(NT, 2),
                      axis=0, keepdims=True).reshape(2, 1)
        lo = jnp.maximum(lo, cl)
        bhi = jnp.minimum(bhi, ch_)

    maskf = (m2 >= lo).astype(f32)
    incl = maskf
    sh = 1
    while sh < L:
        incl = incl + jnp.concatenate(
            [jnp.zeros((2, sh), f32), incl[:, :L - sh]], axis=1)
        sh *= 2
    rank = incl - maskf

    ctxs = []
    for hh in range(2):
        sl = slice(hh * D, (hh + 1) * D)
        q = q2[:, sl]
        k = k2[:, sl]
        v = v2[:, sl]
        rk = rank[hh:hh + 1, :]
        mk = maskf[hh:hh + 1, :]
        p = ((ri == rk) & (mk > 0) & (rk < uf)).astype(f32)

        qr = lax.dot_general(p, q, (((1,), (0,)), ((), ())),
                             preferred_element_type=f32, precision=df)
        st = lax.dot_general(qr, k, (((1,), (1,)), ((), ())),
                             preferred_element_type=f32, precision=df)
        st = st * (1.0 / math.sqrt(D))
        mt = jnp.max(st, axis=1, keepdims=True)
        e = jnp.exp(st - mt)
        a = e / jnp.sum(e, axis=1, keepdims=True)
        upd = lax.dot_general(a, v, (((1,), (0,)), ((), ())),
                              preferred_element_type=f32, precision=df)

        vmean = jnp.sum(v, axis=0, keepdims=True) * (1.0 / L)
        ctxs.append(vmean + lax.dot_general(
            p, upd - vmean, (((0,), (0,)), ((), ())),
            preferred_element_type=f32, precision=df))

    o_ref[...] = jnp.concatenate(ctxs, axis=1).reshape(o_ref.shape)


def kernel(queries, keys, values):
    B, L, H, D = queries.shape
    L_K = keys.shape[1]
    u_samp = min(_FCT * int(np.ceil(np.log(L_K))), L_K)
    U = min(_FCT * int(np.ceil(np.log(L))), L)
    U_pad = max(8, ((U + 63) // 64) * 64)
    CH = 1024

    idx = _np_randint(42, (L, u_samp), 0, L_K)
    cnt_np = np.zeros((L, L_K), np.float32)
    np.add.at(cnt_np, (np.arange(L)[:, None], idx), 1.0)
    cnt = jnp.asarray(cnt_np, dtype=jnp.bfloat16)
    cntT = jnp.asarray(cnt_np.T, dtype=jnp.bfloat16)

    q3 = queries.reshape(B, L, H * D)
    k3 = keys.reshape(B, L, H * D)
    v3 = values.reshape(B, L, H * D)

    body = functools.partial(_pair_kernel, L=L, D=D, U=U, U_pad=U_pad, CH=CH)
    out = pl.pallas_call(
        body,
        grid=(B, H // 2),
        in_specs=[
            pl.BlockSpec((1, L, 2 * D), lambda b, j: (b, 0, j)),
            pl.BlockSpec((1, L, 2 * D), lambda b, j: (b, 0, j)),
            pl.BlockSpec((1, L, 2 * D), lambda b, j: (b, 0, j)),
            pl.BlockSpec(memory_space=pl.ANY),
            pl.BlockSpec(memory_space=pl.ANY),
        ],
        out_specs=pl.BlockSpec((1, L, 2 * D), lambda b, j: (b, 0, j)),
        out_shape=jax.ShapeDtypeStruct((B, L, H * D), jnp.float32),
        scratch_shapes=[
            pltpu.VMEM((L, L_K), jnp.bfloat16),
            pltpu.VMEM((L_K, L), jnp.bfloat16),
            pltpu.SemaphoreType.DMA,
            pltpu.SemaphoreType.DMA,
        ],
        compiler_params=pltpu.CompilerParams(
            dimension_semantics=("parallel", "parallel"),
        ),
    )(q3, k3, v3, cnt, cntT)
    return out.reshape(B, L, H, D)
